# baseline (speedup 1.0000x reference)
import functools

import jax
import jax.numpy as jnp
from jax import lax
from jax.experimental import pallas as pl
from jax.experimental.pallas import tpu as pltpu
from jax.experimental.pallas import tpu_sc as plsc

D = 128
R = 4096
W = 50
V = 10000
NC, NS = 2, 16
NW = NC * NS
R_PER_W = R // NW
CHUNK = 32
NCHUNK = W * 4
NBUF = 10
NGRP = NCHUNK // NBUF

_mesh = plsc.VectorSubcoreMesh(core_axis_name="c", subcore_axis_name="s")


@functools.partial(
    pl.kernel,
    mesh=_mesh,
    out_type=jax.ShapeDtypeStruct((W, R, D), jnp.float32),
    compiler_params=pltpu.CompilerParams(use_tc_tiling_on_sc=True),
    scratch_types=[
        pltpu.VMEM((W, R_PER_W), jnp.int32),
        pltpu.VMEM_SHARED((V, D), jnp.float32),
    ]
    + [pltpu.VMEM((CHUNK, D), jnp.float32) for _ in range(NBUF)]
    + [pltpu.SemaphoreType.DMA for _ in range(2 * NBUF)],
)
def _gather_kernel(pe_hbm, idx_hbm, out_hbm, idx_v, pe_sp, *rest):
    bufs = rest[:NBUF]
    gsems = rest[NBUF:2 * NBUF]
    ssems = rest[2 * NBUF:]

    sid = lax.axis_index("s")
    wid = sid * NC + lax.axis_index("c")
    base = wid * R_PER_W
    pltpu.sync_copy(idx_hbm.at[:, pl.ds(base, R_PER_W)], idx_v)

    rows = 624
    pltpu.sync_copy(
        pe_hbm.at[pl.ds(sid * rows, rows)], pe_sp.at[pl.ds(sid * rows, rows)]
    )

    @pl.when(sid == 0)
    def _():
        pltpu.sync_copy(
            pe_hbm.at[pl.ds(16 * rows, V - 16 * rows)],
            pe_sp.at[pl.ds(16 * rows, V - 16 * rows)],
        )

    plsc.subcore_barrier()

    def gather(c, b):
        w = c // 4
        off = (c % 4) * CHUNK
        src = pe_sp if b % 2 == 0 else pe_hbm
        return pltpu.make_async_copy(
            src.at[idx_v.at[w, pl.ds(off, CHUNK)]], bufs[b], gsems[b]
        )

    def store(c, b):
        w = c // 4
        off = (c % 4) * CHUNK
        return pltpu.make_async_copy(
            bufs[b], out_hbm.at[w, pl.ds(base + off, CHUNK)], ssems[b]
        )

    for b in range(NBUF):
        gather(b, b).start()

    def grp(g, carry):
        c0 = g * NBUF
        for b in range(NBUF):
            gather(c0 + b, b).wait()
            store(c0 + b, b).start()
        for b in range(NBUF):
            store(c0 + b, b).wait()
            gather(c0 + NBUF + b, b).start()
        return carry

    lax.fori_loop(0, NGRP - 1, grp, 0)

    c0 = (NGRP - 1) * NBUF
    for b in range(NBUF):
        gather(c0 + b, b).wait()
        store(c0 + b, b).start()
    for b in range(NBUF):
        store(c0 + b, b).wait()


def kernel(t, pe):
    outT = _gather_kernel(pe, t.T)
    return outT.transpose(1, 0, 2)


# speedup vs baseline: 1.2838x; 1.2838x over previous
import functools

import jax
import jax.numpy as jnp
from jax import lax
from jax.experimental import pallas as pl
from jax.experimental.pallas import tpu as pltpu
from jax.experimental.pallas import tpu_sc as plsc

D = 128
R = 4096
W = 50
V = 10000
NC, NS = 2, 16
NW = NC * NS
R_PER_W = R // NW
CHUNK = 32
NCHUNK = W * 4
NBUF = 10
NGRP = NCHUNK // NBUF

_mesh = plsc.VectorSubcoreMesh(core_axis_name="c", subcore_axis_name="s")


@functools.partial(
    pl.kernel,
    mesh=_mesh,
    out_type=jax.ShapeDtypeStruct((W, R, D), jnp.float32),
    compiler_params=pltpu.CompilerParams(use_tc_tiling_on_sc=True),
    scratch_types=[
        pltpu.VMEM((W, R_PER_W), jnp.int32),
        pltpu.VMEM_SHARED((V, D), jnp.float32),
    ]
    + [pltpu.VMEM((CHUNK, D), jnp.float32) for _ in range(NBUF)]
    + [pltpu.SemaphoreType.DMA for _ in range(2 * NBUF)],
)
def _gather_kernel(pe_hbm, idx_hbm, out_hbm, idx_v, pe_sp, *rest):
    bufs = rest[:NBUF]
    gsems = rest[NBUF:2 * NBUF]
    ssems = rest[2 * NBUF:]

    sid = lax.axis_index("s")
    wid = sid * NC + lax.axis_index("c")
    base = wid * R_PER_W
    pltpu.sync_copy(idx_hbm.at[:, pl.ds(base, R_PER_W)], idx_v)

    for b in range(NBUF):
        pltpu.make_async_copy(
            pe_hbm.at[idx_v.at[b // 4, pl.ds((b % 4) * CHUNK, CHUNK)]],
            bufs[b],
            gsems[b],
        ).start()

    rows = 624
    pltpu.sync_copy(
        pe_hbm.at[pl.ds(sid * rows, rows)], pe_sp.at[pl.ds(sid * rows, rows)]
    )

    @pl.when(sid == 0)
    def _():
        pltpu.sync_copy(
            pe_hbm.at[pl.ds(16 * rows, V - 16 * rows)],
            pe_sp.at[pl.ds(16 * rows, V - 16 * rows)],
        )

    plsc.subcore_barrier()

    def gather(c, b):
        w = c // 4
        off = (c % 4) * CHUNK
        return pltpu.make_async_copy(
            pe_sp.at[idx_v.at[w, pl.ds(off, CHUNK)]], bufs[b], gsems[b]
        )

    def store(c, b):
        w = c // 4
        off = (c % 4) * CHUNK
        return pltpu.make_async_copy(
            bufs[b], out_hbm.at[w, pl.ds(base + off, CHUNK)], ssems[b]
        )

    def grp(g, carry):
        c0 = g * NBUF
        for b in range(NBUF):
            gather(c0 + b, b).wait()
            store(c0 + b, b).start()
        for b in range(NBUF):
            store(c0 + b, b).wait()
            gather(c0 + NBUF + b, b).start()
        return carry

    lax.fori_loop(0, NGRP - 1, grp, 0)

    c0 = (NGRP - 1) * NBUF
    for b in range(NBUF):
        gather(c0 + b, b).wait()
        store(c0 + b, b).start()
    for b in range(NBUF):
        store(c0 + b, b).wait()


def kernel(t, pe):
    outT = _gather_kernel(pe, t.T)
    return outT.transpose(1, 0, 2)
